# baseline (speedup 1.0000x reference)
import functools

import jax
import jax.numpy as jnp
from jax import lax
from jax.experimental import pallas as pl
from jax.experimental.pallas import tpu as pltpu
from jax.experimental.pallas import tpu_sc as plsc

N = 16384
C = 64
K = 8
NC = 2
NS = 16
NW = NC * NS
PPW = N // NW
CHP = 16
PCH = CHP * K
NCHUNK = PPW // CHP
NSUPER = NCHUNK // 2


def _sqrt16(x):
    xs = jnp.maximum(x, jnp.float32(1e-12))
    i = plsc.bitcast(xs, jnp.int32)
    i = jnp.int32(0x5F3759DF) - lax.shift_right_logical(i, 1)
    y = plsc.bitcast(i, jnp.float32)
    for _ in range(2):
        y = y * (jnp.float32(1.5) - jnp.float32(0.5) * xs * y * y)
    return x * y


def _sc_body(mask_hbm, nn_hbm, out_hbm, part_hbm,
             idx_v, cen_v, rows_a, rows_b, pq_a, pq_b, pq_c, pq_d, out_v,
             part_v, sem_a, sem_b):
    cid = lax.axis_index("c")
    sid = lax.axis_index("s")
    wid = sid * NC + cid
    pbase = wid * PPW
    lane = lax.iota(jnp.int32, 16)
    mask2 = mask_hbm.at[0]

    pltpu.sync_copy(nn_hbm.at[pl.ds(pbase * K, PPW * K)], idx_v)
    pltpu.sync_copy(mask2.at[pl.ds(pbase, PPW)], cen_v)

    def start_gather(chunk, rows_v, sem):
        src = mask2.at[idx_v.at[pl.ds(chunk * PCH, PCH)]]
        pltpu.async_copy(src, rows_v, sem)

    def wait_gather(rows_v, sem):
        src = mask2.at[idx_v.at[pl.ds(0, PCH)]]
        pltpu.make_async_copy(src, rows_v, sem).wait()

    def compute_chunk(chunk, rows_v, lacc):
        cpt = chunk * CHP
        cpr = chunk * PCH

        def group16(g, pq, lacc):
            p0 = cpt + g * 2
            jloc = g * 16
            accs = [None] * 16
            for q in range(4):
                cen2 = [cen_v[p0 + pp, pl.ds(16 * q, 16)] for pp in range(2)]
                for j16 in range(16):
                    d = cen2[j16 // K] - rows_v[jloc + j16, pl.ds(16 * q, 16)]
                    sq = d * d
                    accs[j16] = sq if accs[j16] is None else accs[j16] + sq
            for j16 in range(16):
                pq[j16] = accs[j16]
            acc4 = []
            for b in range(4):
                t = plsc.load_gather(
                    pq, [lane, jnp.full((16,), 4 * b, jnp.int32)])
                for c in range(4 * b + 1, 4 * b + 4):
                    t = t + plsc.load_gather(
                        pq, [lane, jnp.full((16,), c, jnp.int32)])
                acc4.append(t)
            acc = (acc4[0] + acc4[1]) + (acc4[2] + acc4[3])
            r = _sqrt16(acc)
            out_v[pl.ds(cpr + g * 16, 16)] = r
            return lacc + r

        def h_body(h, lacc):
            lacc = group16(2 * h, pq_a, lacc)
            lacc = group16(2 * h + 1, pq_b, lacc)
            return lacc

        return lax.fori_loop(0, PCH // 32, h_body, lacc)

    start_gather(0, rows_a, sem_a)
    start_gather(1, rows_b, sem_b)

    def super_body(s, lacc):
        wait_gather(rows_a, sem_a)
        lacc = compute_chunk(2 * s, rows_a, lacc)

        @pl.when(s != NSUPER - 1)
        def _():
            start_gather(2 * s + 2, rows_a, sem_a)

        wait_gather(rows_b, sem_b)
        lacc = compute_chunk(2 * s + 1, rows_b, lacc)

        @pl.when(s != NSUPER - 1)
        def _():
            start_gather(2 * s + 3, rows_b, sem_b)

        return lacc

    lacc = lax.fori_loop(0, NSUPER, super_body, jnp.zeros((16,), jnp.float32))
    part_v[0] = lacc
    pltpu.sync_copy(out_v, out_hbm.at[pl.ds(pbase * K, PPW * K)])
    pltpu.sync_copy(part_v, part_hbm.at[pl.ds(wid, 1)])


_sc_kernel = functools.partial(
    pl.kernel,
    mesh=plsc.VectorSubcoreMesh(core_axis_name="c", subcore_axis_name="s"),
    compiler_params=pltpu.CompilerParams(
        needs_layout_passes=False, use_tc_tiling_on_sc=False
    ),
    out_type=[
        jax.ShapeDtypeStruct((N * K,), jnp.float32),
        jax.ShapeDtypeStruct((NW, 16), jnp.float32),
    ],
    scratch_types=[
        pltpu.VMEM((PPW * K,), jnp.int32),
        pltpu.VMEM((PPW, C), jnp.float32),
        pltpu.VMEM((PCH, C), jnp.float32),
        pltpu.VMEM((PCH, C), jnp.float32),
        pltpu.VMEM((16, 16), jnp.float32),
        pltpu.VMEM((16, 16), jnp.float32),
        pltpu.VMEM((16, 16), jnp.float32),
        pltpu.VMEM((16, 16), jnp.float32),
        pltpu.VMEM((PPW * K,), jnp.float32),
        pltpu.VMEM((1, 16), jnp.float32),
        pltpu.SemaphoreType.DMA,
        pltpu.SemaphoreType.DMA,
    ],
)(_sc_body)


def kernel(mask, nn_ind):
    per_flat, parts = _sc_kernel(mask, nn_ind.reshape(N * K))
    loss = jnp.sum(parts) * jnp.float32(1.0 / (N * K))
    return loss, per_flat.reshape(1, N, K)


# speedup vs baseline: 1.0353x; 1.0353x over previous
import functools

import jax
import jax.numpy as jnp
from jax import lax
from jax.experimental import pallas as pl
from jax.experimental.pallas import tpu as pltpu
from jax.experimental.pallas import tpu_sc as plsc

N = 16384
C = 64
K = 8
NC = 2
NS = 16
NW = NC * NS
PPW = N // NW
CHP = 16
PCH = CHP * K
NCHUNK = PPW // CHP
NSUPER = NCHUNK // 2


def _sqrt16(x):
    xs = jnp.maximum(x, jnp.float32(1e-12))
    i = plsc.bitcast(xs, jnp.int32)
    i = jnp.int32(0x5F3759DF) - lax.shift_right_logical(i, 1)
    y = plsc.bitcast(i, jnp.float32)
    for _ in range(2):
        y = y * (jnp.float32(1.5) - jnp.float32(0.5) * xs * y * y)
    return x * y


def _sc_body(mask_hbm, nn_hbm, out_hbm, part_hbm,
             idx_v, cen_v, rows_a, rows_b, pq_a, pq_b, pq_c, pq_d, out_v,
             part_v, sem_a, sem_b):
    cid = lax.axis_index("c")
    sid = lax.axis_index("s")
    wid = sid * NC + cid
    pbase = wid * PPW
    lane = lax.iota(jnp.int32, 16)
    mask2 = mask_hbm.at[0]

    pltpu.sync_copy(nn_hbm.at[pl.ds(pbase * K, PPW * K)], idx_v)
    pltpu.sync_copy(mask2.at[pl.ds(pbase, PPW)], cen_v)

    def start_gather(chunk, rows_v, sem):
        src = mask2.at[idx_v.at[pl.ds(chunk * PCH, PCH)]]
        pltpu.async_copy(src, rows_v, sem)

    def wait_gather(rows_v, sem):
        src = mask2.at[idx_v.at[pl.ds(0, PCH)]]
        pltpu.make_async_copy(src, rows_v, sem).wait()

    def compute_chunk(chunk, rows_v, lacc):
        cpt = chunk * CHP
        cpr = chunk * PCH

        def group16(g, pq, lacc):
            p0 = cpt + g * 2
            jloc = g * 16
            accs = [None] * 16
            for q in range(4):
                cen2 = [cen_v[p0 + pp, pl.ds(16 * q, 16)] for pp in range(2)]
                for j16 in range(16):
                    d = cen2[j16 // K] - rows_v[jloc + j16, pl.ds(16 * q, 16)]
                    sq = d * d
                    accs[j16] = sq if accs[j16] is None else accs[j16] + sq
            acc = jnp.zeros((16,), jnp.float32)
            for j16 in range(16):
                acc = jnp.where(lane == j16, jnp.sum(accs[j16]), acc)
            r = _sqrt16(acc)
            out_v[pl.ds(cpr + g * 16, 16)] = r
            return lacc + r

        def h_body(h, lacc):
            return group16(h, pq_a, lacc)

        return lax.fori_loop(0, PCH // 16, h_body, lacc)

    start_gather(0, rows_a, sem_a)
    start_gather(1, rows_b, sem_b)

    def super_body(s, lacc):
        wait_gather(rows_a, sem_a)
        lacc = compute_chunk(2 * s, rows_a, lacc)

        @pl.when(s != NSUPER - 1)
        def _():
            start_gather(2 * s + 2, rows_a, sem_a)

        wait_gather(rows_b, sem_b)
        lacc = compute_chunk(2 * s + 1, rows_b, lacc)

        @pl.when(s != NSUPER - 1)
        def _():
            start_gather(2 * s + 3, rows_b, sem_b)

        return lacc

    lacc = lax.fori_loop(0, NSUPER, super_body, jnp.zeros((16,), jnp.float32))
    part_v[0] = lacc
    pltpu.sync_copy(out_v, out_hbm.at[pl.ds(pbase * K, PPW * K)])
    pltpu.sync_copy(part_v, part_hbm.at[pl.ds(wid, 1)])


_sc_kernel = functools.partial(
    pl.kernel,
    mesh=plsc.VectorSubcoreMesh(core_axis_name="c", subcore_axis_name="s"),
    compiler_params=pltpu.CompilerParams(
        needs_layout_passes=False, use_tc_tiling_on_sc=False
    ),
    out_type=[
        jax.ShapeDtypeStruct((N * K,), jnp.float32),
        jax.ShapeDtypeStruct((NW, 16), jnp.float32),
    ],
    scratch_types=[
        pltpu.VMEM((PPW * K,), jnp.int32),
        pltpu.VMEM((PPW, C), jnp.float32),
        pltpu.VMEM((PCH, C), jnp.float32),
        pltpu.VMEM((PCH, C), jnp.float32),
        pltpu.VMEM((16, 16), jnp.float32),
        pltpu.VMEM((16, 16), jnp.float32),
        pltpu.VMEM((16, 16), jnp.float32),
        pltpu.VMEM((16, 16), jnp.float32),
        pltpu.VMEM((PPW * K,), jnp.float32),
        pltpu.VMEM((1, 16), jnp.float32),
        pltpu.SemaphoreType.DMA,
        pltpu.SemaphoreType.DMA,
    ],
)(_sc_body)


def kernel(mask, nn_ind):
    per_flat, parts = _sc_kernel(mask, nn_ind.reshape(N * K))
    loss = jnp.sum(parts) * jnp.float32(1.0 / (N * K))
    return loss, per_flat.reshape(1, N, K)


# speedup vs baseline: 1.0426x; 1.0070x over previous
import functools

import jax
import jax.numpy as jnp
from jax import lax
from jax.experimental import pallas as pl
from jax.experimental.pallas import tpu as pltpu
from jax.experimental.pallas import tpu_sc as plsc

N = 16384
C = 64
K = 8
NC = 2
NS = 16
NW = NC * NS
PPW = N // NW
CHP = 16
PCH = CHP * K
NCHUNK = PPW // CHP
NSUPER = NCHUNK // 2


def _sqrt16(x):
    xs = jnp.maximum(x, jnp.float32(1e-12))
    i = plsc.bitcast(xs, jnp.int32)
    i = jnp.int32(0x5F3759DF) - lax.shift_right_logical(i, 1)
    y = plsc.bitcast(i, jnp.float32)
    for _ in range(2):
        y = y * (jnp.float32(1.5) - jnp.float32(0.5) * xs * y * y)
    return x * y


def _sc_body(mask_hbm, nn_hbm, out_hbm, part_hbm,
             idx_v, cen_v, rows_a, rows_b, pq_a, pq_b, pq_c, pq_d, out_v,
             part_v, sem_a, sem_b):
    cid = lax.axis_index("c")
    sid = lax.axis_index("s")
    wid = sid * NC + cid
    pbase = wid * PPW
    lane = lax.iota(jnp.int32, 16)
    mask2 = mask_hbm.at[0]

    pltpu.sync_copy(nn_hbm.at[pl.ds(wid * NCHUNK, NCHUNK)], idx_v)
    pltpu.sync_copy(mask2.at[pl.ds(pbase, PPW)], cen_v)

    def start_gather(chunk, rows_v, sem):
        src = mask2.at[idx_v.at[chunk]]
        pltpu.async_copy(src, rows_v, sem)

    def wait_gather(rows_v, sem):
        src = mask2.at[idx_v.at[0]]
        pltpu.make_async_copy(src, rows_v, sem).wait()

    def compute_chunk(chunk, rows_v, lacc):
        cpt = chunk * CHP
        cpr = chunk * PCH

        def group16(g, pq, lacc):
            p0 = cpt + g * 2
            jloc = g * 16
            accs = [None] * 16
            for q in range(4):
                cen2 = [cen_v[p0 + pp, pl.ds(16 * q, 16)] for pp in range(2)]
                for j16 in range(16):
                    d = cen2[j16 // K] - rows_v[jloc + j16, pl.ds(16 * q, 16)]
                    sq = d * d
                    accs[j16] = sq if accs[j16] is None else accs[j16] + sq
            acc = jnp.zeros((16,), jnp.float32)
            for j16 in range(16):
                acc = jnp.where(lane == j16, jnp.sum(accs[j16]), acc)
            r = _sqrt16(acc)
            out_v[chunk, pl.ds(g * 16, 16)] = r
            return lacc + r

        def h_body(h, lacc):
            return group16(h, pq_a, lacc)

        return lax.fori_loop(0, PCH // 16, h_body, lacc)

    start_gather(0, rows_a, sem_a)
    start_gather(1, rows_b, sem_b)

    def super_body(s, lacc):
        wait_gather(rows_a, sem_a)
        lacc = compute_chunk(2 * s, rows_a, lacc)

        @pl.when(s != NSUPER - 1)
        def _():
            start_gather(2 * s + 2, rows_a, sem_a)

        wait_gather(rows_b, sem_b)
        lacc = compute_chunk(2 * s + 1, rows_b, lacc)

        @pl.when(s != NSUPER - 1)
        def _():
            start_gather(2 * s + 3, rows_b, sem_b)

        return lacc

    lacc = lax.fori_loop(0, NSUPER, super_body, jnp.zeros((16,), jnp.float32))
    part_v[0] = lacc
    pltpu.sync_copy(out_v, out_hbm.at[pl.ds(wid * NCHUNK, NCHUNK)])
    pltpu.sync_copy(part_v, part_hbm.at[pl.ds(wid, 1)])


_sc_kernel = functools.partial(
    pl.kernel,
    mesh=plsc.VectorSubcoreMesh(core_axis_name="c", subcore_axis_name="s"),
    compiler_params=pltpu.CompilerParams(
        needs_layout_passes=False, use_tc_tiling_on_sc=False
    ),
    out_type=[
        jax.ShapeDtypeStruct((N * K // PCH, PCH), jnp.float32),
        jax.ShapeDtypeStruct((NW, 16), jnp.float32),
    ],
    scratch_types=[
        pltpu.VMEM((NCHUNK, PCH), jnp.int32),
        pltpu.VMEM((PPW, C), jnp.float32),
        pltpu.VMEM((PCH, C), jnp.float32),
        pltpu.VMEM((PCH, C), jnp.float32),
        pltpu.VMEM((16, 16), jnp.float32),
        pltpu.VMEM((16, 16), jnp.float32),
        pltpu.VMEM((16, 16), jnp.float32),
        pltpu.VMEM((16, 16), jnp.float32),
        pltpu.VMEM((NCHUNK, PCH), jnp.float32),
        pltpu.VMEM((1, 16), jnp.float32),
        pltpu.SemaphoreType.DMA,
        pltpu.SemaphoreType.DMA,
    ],
)(_sc_body)


def kernel(mask, nn_ind):
    per2d, parts = _sc_kernel(mask, nn_ind.reshape(N * K // PCH, PCH))
    loss = jnp.sum(parts) * jnp.float32(1.0 / (N * K))
    return loss, per2d.reshape(1, N, K)


# speedup vs baseline: 1.0432x; 1.0006x over previous
import functools

import jax
import jax.numpy as jnp
from jax import lax
from jax.experimental import pallas as pl
from jax.experimental.pallas import tpu as pltpu
from jax.experimental.pallas import tpu_sc as plsc

N = 16384
C = 64
K = 8
NC = 2
NS = 16
NW = NC * NS
PPW = N // NW
CHP = 16
PCH = CHP * K
NCHUNK = PPW // CHP
NSUPER = NCHUNK // 2


def _sqrt16(x):
    xs = jnp.maximum(x, jnp.float32(1e-12))
    i = plsc.bitcast(xs, jnp.int32)
    i = jnp.int32(0x5F3759DF) - lax.shift_right_logical(i, 1)
    y = plsc.bitcast(i, jnp.float32)
    for _ in range(2):
        y = y * (jnp.float32(1.5) - jnp.float32(0.5) * xs * y * y)
    return x * y


def _sc_body(mask_hbm, nn_hbm, out_hbm, part_hbm,
             idx_v, cen_v, rows_a, rows_b, pq_a, pq_b, pq_c, pq_d, out_v,
             part_v, sem_a, sem_b):
    cid = lax.axis_index("c")
    sid = lax.axis_index("s")
    wid = sid * NC + cid
    pbase = wid * PPW
    lane = lax.iota(jnp.int32, 16)
    mask2 = mask_hbm.at[0]

    pltpu.sync_copy(nn_hbm.at[pl.ds(wid * NCHUNK, NCHUNK)], idx_v)
    pltpu.sync_copy(mask2.at[pl.ds(pbase, PPW)], cen_v)

    def start_gather(chunk, rows_v, sem):
        src = mask2.at[idx_v.at[chunk]]
        pltpu.async_copy(src, rows_v, sem)

    def wait_gather(rows_v, sem):
        src = mask2.at[idx_v.at[0]]
        pltpu.make_async_copy(src, rows_v, sem).wait()

    def compute_chunk(chunk, rows_v, lacc):
        cpt = chunk * CHP
        cpr = chunk * PCH

        def group16(g, pq, lacc):
            p0 = cpt + g * 2
            jloc = g * 16
            accs = [None] * 16
            for q in range(4):
                cen2 = [cen_v[p0 + pp, pl.ds(16 * q, 16)] for pp in range(2)]
                for j16 in range(16):
                    d = cen2[j16 // K] - rows_v[jloc + j16, pl.ds(16 * q, 16)]
                    sq = d * d
                    accs[j16] = sq if accs[j16] is None else accs[j16] + sq
            acc = jnp.zeros((16,), jnp.float32)
            for j16 in range(16):
                acc = jnp.where(lane == j16, jnp.sum(accs[j16]), acc)
            r = _sqrt16(acc)
            out_v[chunk, pl.ds(g * 16, 16)] = r
            return lacc + r

        def h_body(h, lacc):
            lacc = group16(2 * h, pq_a, lacc)
            lacc = group16(2 * h + 1, pq_b, lacc)
            return lacc

        return lax.fori_loop(0, PCH // 32, h_body, lacc)

    start_gather(0, rows_a, sem_a)
    start_gather(1, rows_b, sem_b)

    def super_body(s, lacc):
        wait_gather(rows_a, sem_a)
        lacc = compute_chunk(2 * s, rows_a, lacc)

        @pl.when(s != NSUPER - 1)
        def _():
            start_gather(2 * s + 2, rows_a, sem_a)

        wait_gather(rows_b, sem_b)
        lacc = compute_chunk(2 * s + 1, rows_b, lacc)

        @pl.when(s != NSUPER - 1)
        def _():
            start_gather(2 * s + 3, rows_b, sem_b)

        return lacc

    lacc = lax.fori_loop(0, NSUPER, super_body, jnp.zeros((16,), jnp.float32))
    part_v[0] = lacc
    pltpu.sync_copy(out_v, out_hbm.at[pl.ds(wid * NCHUNK, NCHUNK)])
    pltpu.sync_copy(part_v, part_hbm.at[pl.ds(wid, 1)])


_sc_kernel = functools.partial(
    pl.kernel,
    mesh=plsc.VectorSubcoreMesh(core_axis_name="c", subcore_axis_name="s"),
    compiler_params=pltpu.CompilerParams(
        needs_layout_passes=False, use_tc_tiling_on_sc=False
    ),
    out_type=[
        jax.ShapeDtypeStruct((N * K // PCH, PCH), jnp.float32),
        jax.ShapeDtypeStruct((NW, 16), jnp.float32),
    ],
    scratch_types=[
        pltpu.VMEM((NCHUNK, PCH), jnp.int32),
        pltpu.VMEM((PPW, C), jnp.float32),
        pltpu.VMEM((PCH, C), jnp.float32),
        pltpu.VMEM((PCH, C), jnp.float32),
        pltpu.VMEM((16, 16), jnp.float32),
        pltpu.VMEM((16, 16), jnp.float32),
        pltpu.VMEM((16, 16), jnp.float32),
        pltpu.VMEM((16, 16), jnp.float32),
        pltpu.VMEM((NCHUNK, PCH), jnp.float32),
        pltpu.VMEM((1, 16), jnp.float32),
        pltpu.SemaphoreType.DMA,
        pltpu.SemaphoreType.DMA,
    ],
)(_sc_body)


def kernel(mask, nn_ind):
    per2d, parts = _sc_kernel(mask, nn_ind.reshape(N * K // PCH, PCH))
    loss = jnp.sum(parts) * jnp.float32(1.0 / (N * K))
    return loss, per2d.reshape(1, N, K)
